# baseline (speedup 1.0000x reference)
import functools
import math

import jax
import jax.numpy as jnp
from jax import lax
from jax.experimental import pallas as pl
from jax.experimental.pallas import tpu as pltpu
from jax.experimental.pallas import tpu_sc as plsc

_B, _S, _D = 16, 4096, 1024
_TARGET = 512
_DIFF = _S % _TARGET
_STEP = math.ceil((_S - _DIFF) / _TARGET)
_OFF = _DIFF // 2

_R = _B * _TARGET
_NW = 32
_RPW = _R // _NW
_CHUNK = 64
_NCHUNK = _RPW // _CHUNK

_mesh = plsc.VectorSubcoreMesh(core_axis_name="c", subcore_axis_name="s")


@functools.partial(
    pl.kernel,
    mesh=_mesh,
    out_type=jax.ShapeDtypeStruct((_R, _D), jnp.float32),
    scratch_types=[
        pltpu.SemaphoreType.DMA,
    ],
)
def _downsample(x_hbm, out_hbm, sem):
    wid = lax.axis_index("s") * 2 + lax.axis_index("c")
    base = wid * _RPW

    copies = []
    for g in range(_NCHUNK):
        rb = base + g * _CHUNK
        copies.append(pltpu.async_copy(
            x_hbm.at[pl.ds(rb, _CHUNK), pl.ds(_OFF * _D, _D)],
            out_hbm.at[pl.ds(rb, _CHUNK)], sem))
    for c in copies:
        c.wait()


def kernel(x):
    xw = x.reshape(_R, _STEP * _D)
    out = _downsample(xw)
    return out.reshape(_B, _TARGET, _D)


# speedup vs baseline: 29.2273x; 29.2273x over previous
import functools
import math

import jax
import jax.numpy as jnp
from jax import lax
from jax.experimental import pallas as pl
from jax.experimental.pallas import tpu as pltpu
from jax.experimental.pallas import tpu_sc as plsc

_B, _S, _D = 16, 4096, 1024
_TARGET = 512
_DIFF = _S % _TARGET
_STEP = math.ceil((_S - _DIFF) / _TARGET)
_OFF = _DIFF // 2

_R = _B * _TARGET
_NW = 32
_RPW = _R // _NW
_CHUNK = 32
_NCHUNK = _RPW // _CHUNK
_NBUF = 3
_LANES = 16

_mesh = plsc.VectorSubcoreMesh(core_axis_name="c", subcore_axis_name="s")


@functools.partial(
    pl.kernel,
    mesh=_mesh,
    out_type=jax.ShapeDtypeStruct((_R, _D), jnp.float32),
    scratch_types=[
        pltpu.VMEM((_RPW,), jnp.int32),
        pltpu.VMEM((_CHUNK, _D), jnp.float32),
        pltpu.VMEM((_CHUNK, _D), jnp.float32),
        pltpu.VMEM((_CHUNK, _D), jnp.float32),
        pltpu.SemaphoreType.DMA,
        pltpu.SemaphoreType.DMA,
        pltpu.SemaphoreType.DMA,
        pltpu.SemaphoreType.DMA,
        pltpu.SemaphoreType.DMA,
        pltpu.SemaphoreType.DMA,
    ],
)
def _downsample(x_hbm, out_hbm, idx_v, rows0, rows1, rows2,
                gsem0, gsem1, gsem2, ssem0, ssem1, ssem2):
    wid = lax.axis_index("s") * 2 + lax.axis_index("c")
    base = wid * _RPW

    lane = lax.iota(jnp.int32, _LANES)
    for j in range(_RPW // _LANES):
        idx_v[pl.ds(j * _LANES, _LANES)] = (
            (lane + (base + j * _LANES)) * _STEP + _OFF)

    bufs = (rows0, rows1, rows2)
    gsems = (gsem0, gsem1, gsem2)
    ssems = (ssem0, ssem1, ssem2)

    def gather(g):
        return pltpu.async_copy(
            x_hbm.at[idx_v.at[pl.ds(g * _CHUNK, _CHUNK)]],
            bufs[g % _NBUF], gsems[g % _NBUF])

    def scatter(g):
        return pltpu.async_copy(
            bufs[g % _NBUF], out_hbm.at[pl.ds(base + g * _CHUNK, _CHUNK)],
            ssems[g % _NBUF])

    gathers = [None] * _NCHUNK
    scatters = [None] * _NCHUNK
    for g in range(_NCHUNK + 1):
        if g >= 1:
            gathers[g - 1].wait()
            scatters[g - 1] = scatter(g - 1)
        if g < _NCHUNK:
            if g >= _NBUF:
                scatters[g - _NBUF].wait()
            gathers[g] = gather(g)
    for g in range(_NCHUNK - _NBUF, _NCHUNK):
        scatters[g].wait()


def kernel(x):
    xf = x.reshape(_B * _S, _D)
    out = _downsample(xf)
    return out.reshape(_B, _TARGET, _D)
